# Initial kernel scaffold; baseline (speedup 1.0000x reference)
import jax
import jax.numpy as jnp
from jax.experimental import pallas as pl


def kernel(inputs, embedding):
    raise NotImplementedError("write your pallas kernel here")


# speedup vs baseline: 1.3748x; 1.3748x over previous
import jax
import jax.numpy as jnp
from jax.experimental import pallas as pl


def _add_kernel(x_ref, e1_ref, e2_ref, o_ref):
    e = jnp.concatenate([e1_ref[1:, :], e2_ref[:1, :]], axis=0)
    o_ref[...] = x_ref[...] + e[None, :, :]


def kernel(inputs, embedding):
    B, T, D = inputs.shape
    BT = 512
    grid = (T // BT, B)
    return pl.pallas_call(
        _add_kernel,
        grid=grid,
        in_specs=[
            pl.BlockSpec((1, BT, D), lambda t, b: (b, t, 0)),
            pl.BlockSpec((BT, D), lambda t, b: (t, 0)),
            pl.BlockSpec((BT, D), lambda t, b: (t + 1, 0)),
        ],
        out_specs=pl.BlockSpec((1, BT, D), lambda t, b: (b, t, 0)),
        out_shape=jax.ShapeDtypeStruct((B, T, D), inputs.dtype),
    )(inputs, embedding, embedding)


# speedup vs baseline: 1.6690x; 1.2140x over previous
import jax
import jax.numpy as jnp
from jax.experimental import pallas as pl


def _add_kernel(x_ref, e1_ref, e2_ref, o_ref):
    e = jnp.concatenate([e1_ref[1:, :], e2_ref[:1, :]], axis=0)
    o_ref[...] = x_ref[...] + e[None, :, :]


def kernel(inputs, embedding):
    B, T, D = inputs.shape
    BT = 1024
    grid = (T // BT, B)
    return pl.pallas_call(
        _add_kernel,
        grid=grid,
        in_specs=[
            pl.BlockSpec((1, BT, D), lambda t, b: (b, t, 0)),
            pl.BlockSpec((BT, D), lambda t, b: (t, 0)),
            pl.BlockSpec((8, D), lambda t, b: ((t + 1) * (BT // 8), 0)),
        ],
        out_specs=pl.BlockSpec((1, BT, D), lambda t, b: (b, t, 0)),
        out_shape=jax.ShapeDtypeStruct((B, T, D), inputs.dtype),
    )(inputs, embedding, embedding)


# speedup vs baseline: 1.7397x; 1.0424x over previous
import jax
import jax.numpy as jnp
from jax.experimental import pallas as pl


def _add_kernel(x_ref, e1_ref, e2_ref, o_ref):
    e = jnp.concatenate([e1_ref[1:, :], e2_ref[:1, :]], axis=0)
    o_ref[...] = x_ref[...] + e[None, :, :]


def kernel(inputs, embedding):
    B, T, D = inputs.shape
    BT = 2048
    grid = (T // BT, B)
    return pl.pallas_call(
        _add_kernel,
        grid=grid,
        in_specs=[
            pl.BlockSpec((1, BT, D), lambda t, b: (b, t, 0)),
            pl.BlockSpec((BT, D), lambda t, b: (t, 0)),
            pl.BlockSpec((8, D), lambda t, b: ((t + 1) * (BT // 8), 0)),
        ],
        out_specs=pl.BlockSpec((1, BT, D), lambda t, b: (b, t, 0)),
        out_shape=jax.ShapeDtypeStruct((B, T, D), inputs.dtype),
    )(inputs, embedding, embedding)
